# Initial kernel scaffold; baseline (speedup 1.0000x reference)
import jax
import jax.numpy as jnp
from jax.experimental import pallas as pl

_B, _S, _D = 1, 2048, 768
_H, _QD = 8, 256
_PK = 128
_SUBQ = _QD // 2
_TOPK = 8
_KP = 8
_EUSE = 256

_TB = 256


def _topk_cols(s, k, width):
    iota = jax.lax.broadcasted_iota(jnp.int32, s.shape, 1)
    vals = s
    out_v, out_i = [], []
    for _ in range(k):
        m = jnp.max(vals, axis=1, keepdims=True)
        cand = jnp.where(vals == m, iota, width)
        idx = jnp.min(cand, axis=1, keepdims=True)
        out_v.append(m)
        out_i.append(idx)
        vals = jnp.where(iota == idx, -jnp.inf, vals)
    return out_v, out_i


def _l2n_rows(v):
    n = jnp.sqrt(jnp.sum(v * v, axis=1, keepdims=True))
    return v / jnp.maximum(n, 1e-12)


def _peer_body(x_ref, wq_ref, bq_ref, g_ref, b_ref, sk0_ref, sk1_ref,
               ed_ref, eu_ref, o_ref):
    f32 = jnp.float32
    hp = jax.lax.Precision.HIGHEST
    xb = x_ref[...]

    q = jnp.dot(xb, wq_ref[...], preferred_element_type=f32,
                precision=hp) + bq_ref[...]

    sk0n = _l2n_rows(sk0_ref[...])
    sk1n = _l2n_rows(sk1_ref[...])

    dense_d = jax.lax.dot_general(
        xb, ed_ref[...], (((1,), (1,)), ((), ())),
        preferred_element_type=f32, precision=hp)

    iota_e = jax.lax.broadcasted_iota(jnp.int32, (_TB, _EUSE), 1)
    iota8 = jax.lax.broadcasted_iota(jnp.int32, (_TB, _KP), 1)
    acc = jnp.zeros((_TB, _EUSE), f32)

    for h in range(_H):
        qh = q[:, h * _QD:(h + 1) * _QD]
        mu = jnp.mean(qh, axis=1, keepdims=True)
        qc = qh - mu
        var = jnp.mean(qc * qc, axis=1, keepdims=True)
        qn = qc * jax.lax.rsqrt(var + 1e-5) * g_ref[...] + b_ref[...]

        q0n = _l2n_rows(qn[:, :_SUBQ])
        q1n = _l2n_rows(qn[:, _SUBQ:])

        s0 = jax.lax.dot_general(q0n, sk0n, (((1,), (1,)), ((), ())),
                                 preferred_element_type=f32, precision=hp)
        s1 = jax.lax.dot_general(q1n, sk1n, (((1,), (1,)), ((), ())),
                                 preferred_element_type=f32, precision=hp)

        ts0, ti0 = _topk_cols(s0, _KP, _PK)
        ts1, ti1 = _topk_cols(s1, _KP, _PK)
        ts1c = jnp.concatenate(ts1, axis=1)
        ti0c = jnp.concatenate(ti0, axis=1)
        ti1c = jnp.concatenate(ti1, axis=1)

        comb = jnp.concatenate([ts0[a] + ts1c for a in range(_KP)],
                               axis=1)
        fs, fpos = _topk_cols(comb, _TOPK, _KP * _KP)

        m = fs[0]
        es = [jnp.exp(v - m) for v in fs]
        zden = es[0]
        for v in es[1:]:
            zden = zden + v
        inv_z = 1.0 / zden

        for k in range(_TOPK):
            pos = fpos[k]
            a = jax.lax.shift_right_logical(pos, 3)
            bsub = pos - (a << 3)
            i0 = jnp.sum(jnp.where(iota8 == a, ti0c, 0), axis=1,
                         keepdims=True)
            i1 = jnp.sum(jnp.where(iota8 == bsub, ti1c, 0), axis=1,
                         keepdims=True)
            eidx = i0 + i1
            gk = es[k] * inv_z

            onehot = eidx == iota_e
            hk = jnp.sum(jnp.where(onehot, dense_d, 0.0), axis=1,
                         keepdims=True)
            hk = 0.5 * hk * (1.0 + jax.lax.erf(hk * 0.7071067811865476))
            hk = hk * gk
            acc = acc + jnp.where(onehot, hk, 0.0)

    o_ref[...] = jnp.dot(acc, eu_ref[...], preferred_element_type=f32,
                         precision=hp)


@jax.jit
def kernel(hidden_states, Wq, bq, ln_gamma, ln_beta, sub_keys_0, sub_keys_1,
           expert_down, expert_up):
    x = hidden_states.reshape(_S, _D)
    out = pl.pallas_call(
        _peer_body,
        grid=(_S // _TB,),
        in_specs=[
            pl.BlockSpec((_TB, _D), lambda i: (i, 0)),
            pl.BlockSpec((_D, _H * _QD), lambda i: (0, 0)),
            pl.BlockSpec((1, _H * _QD), lambda i: (0, 0)),
            pl.BlockSpec((1, _QD), lambda i: (0, 0)),
            pl.BlockSpec((1, _QD), lambda i: (0, 0)),
            pl.BlockSpec((_PK, _SUBQ), lambda i: (0, 0)),
            pl.BlockSpec((_PK, _SUBQ), lambda i: (0, 0)),
            pl.BlockSpec((_EUSE, _D), lambda i: (0, 0)),
            pl.BlockSpec((_EUSE, _D), lambda i: (0, 0)),
        ],
        out_specs=pl.BlockSpec((_TB, _D), lambda i: (i, 0)),
        out_shape=jax.ShapeDtypeStruct((_S, _D), jnp.float32),
    )(x, Wq, bq.reshape(1, _H * _QD), ln_gamma.reshape(1, _QD),
      ln_beta.reshape(1, _QD), sub_keys_0, sub_keys_1, expert_down, expert_up)
    return out.reshape(_B, _S, _D)


# speedup vs baseline: 6.4347x; 6.4347x over previous
import jax
import jax.numpy as jnp
from jax.experimental import pallas as pl

_B, _S, _D = 1, 2048, 768
_H, _QD = 8, 256
_PK = 128
_SUBQ = _QD // 2
_TOPK = 8
_KP = 8
_EUSE = 256

_TB = 256


def _topk_cols(s, k, width):
    iota = jax.lax.broadcasted_iota(jnp.int32, s.shape, 1)
    vals = s
    out_v, out_i = [], []
    for _ in range(k):
        m = jnp.max(vals, axis=1, keepdims=True)
        cand = jnp.where(vals == m, iota, width)
        idx = jnp.min(cand, axis=1, keepdims=True)
        out_v.append(m)
        out_i.append(idx)
        vals = jnp.where(iota == idx, -jnp.inf, vals)
    return out_v, out_i


def _l2n_rows(v):
    n = jnp.sqrt(jnp.sum(v * v, axis=1, keepdims=True))
    return v / jnp.maximum(n, 1e-12)


def _peer_body(x_ref, wq_ref, bq_ref, g_ref, b_ref, sk0_ref, sk1_ref,
               ed_ref, eu_ref, o_ref):
    f32 = jnp.float32
    hp = jax.lax.Precision.DEFAULT
    xb = x_ref[...]

    q = jnp.dot(xb, wq_ref[...], preferred_element_type=f32,
                precision=hp) + bq_ref[...]

    sk0n = _l2n_rows(sk0_ref[...])
    sk1n = _l2n_rows(sk1_ref[...])

    dense_d = jax.lax.dot_general(
        xb, ed_ref[...], (((1,), (1,)), ((), ())),
        preferred_element_type=f32, precision=hp)

    iota_e = jax.lax.broadcasted_iota(jnp.int32, (_TB, _EUSE), 1)
    iota8 = jax.lax.broadcasted_iota(jnp.int32, (_TB, _KP), 1)
    acc = jnp.zeros((_TB, _EUSE), f32)

    for h in range(_H):
        qh = q[:, h * _QD:(h + 1) * _QD]
        mu = jnp.mean(qh, axis=1, keepdims=True)
        qc = qh - mu
        var = jnp.mean(qc * qc, axis=1, keepdims=True)
        qn = qc * jax.lax.rsqrt(var + 1e-5) * g_ref[...] + b_ref[...]

        q0n = _l2n_rows(qn[:, :_SUBQ])
        q1n = _l2n_rows(qn[:, _SUBQ:])

        s0 = jax.lax.dot_general(q0n, sk0n, (((1,), (1,)), ((), ())),
                                 preferred_element_type=f32, precision=hp)
        s1 = jax.lax.dot_general(q1n, sk1n, (((1,), (1,)), ((), ())),
                                 preferred_element_type=f32, precision=hp)

        ts0, ti0 = _topk_cols(s0, _KP, _PK)
        ts1, ti1 = _topk_cols(s1, _KP, _PK)
        ts1c = jnp.concatenate(ts1, axis=1)
        ti0c = jnp.concatenate(ti0, axis=1)
        ti1c = jnp.concatenate(ti1, axis=1)

        comb = jnp.concatenate([ts0[a] + ts1c for a in range(_KP)],
                               axis=1)
        fs, fpos = _topk_cols(comb, _TOPK, _KP * _KP)

        m = fs[0]
        es = [jnp.exp(v - m) for v in fs]
        zden = es[0]
        for v in es[1:]:
            zden = zden + v
        inv_z = 1.0 / zden

        for k in range(_TOPK):
            pos = fpos[k]
            a = jax.lax.shift_right_logical(pos, 3)
            bsub = pos - (a << 3)
            i0 = jnp.sum(jnp.where(iota8 == a, ti0c, 0), axis=1,
                         keepdims=True)
            i1 = jnp.sum(jnp.where(iota8 == bsub, ti1c, 0), axis=1,
                         keepdims=True)
            eidx = i0 + i1
            gk = es[k] * inv_z

            onehot = eidx == iota_e
            hk = jnp.sum(jnp.where(onehot, dense_d, 0.0), axis=1,
                         keepdims=True)
            hk = 0.5 * hk * (1.0 + jax.lax.erf(hk * 0.7071067811865476))
            hk = hk * gk
            acc = acc + jnp.where(onehot, hk, 0.0)

    o_ref[...] = jnp.dot(acc, eu_ref[...], preferred_element_type=f32,
                         precision=hp)


@jax.jit
def kernel(hidden_states, Wq, bq, ln_gamma, ln_beta, sub_keys_0, sub_keys_1,
           expert_down, expert_up):
    x = hidden_states.reshape(_S, _D)
    out = pl.pallas_call(
        _peer_body,
        grid=(_S // _TB,),
        in_specs=[
            pl.BlockSpec((_TB, _D), lambda i: (i, 0)),
            pl.BlockSpec((_D, _H * _QD), lambda i: (0, 0)),
            pl.BlockSpec((1, _H * _QD), lambda i: (0, 0)),
            pl.BlockSpec((1, _QD), lambda i: (0, 0)),
            pl.BlockSpec((1, _QD), lambda i: (0, 0)),
            pl.BlockSpec((_PK, _SUBQ), lambda i: (0, 0)),
            pl.BlockSpec((_PK, _SUBQ), lambda i: (0, 0)),
            pl.BlockSpec((_EUSE, _D), lambda i: (0, 0)),
            pl.BlockSpec((_EUSE, _D), lambda i: (0, 0)),
        ],
        out_specs=pl.BlockSpec((_TB, _D), lambda i: (i, 0)),
        out_shape=jax.ShapeDtypeStruct((_S, _D), jnp.float32),
    )(x, Wq, bq.reshape(1, _H * _QD), ln_gamma.reshape(1, _QD),
      ln_beta.reshape(1, _QD), sub_keys_0, sub_keys_1, expert_down, expert_up)
    return out.reshape(_B, _S, _D)


# speedup vs baseline: 30.1798x; 4.6901x over previous
import jax
import jax.numpy as jnp
from jax.experimental import pallas as pl

_B, _S, _D = 1, 2048, 768
_H, _QD = 8, 256
_PK = 128
_SUBQ = _QD // 2
_TOPK = 8
_KP = 8
_EUSE = 256

_TB = 256

_NN = (((1,), (0,)), ((), ()))
_NT = (((1,), (1,)), ((), ()))


def _topk_rows(s, k, width):
    iota = jax.lax.broadcasted_iota(jnp.int32, s.shape, 0)
    vals = s
    out_v, out_i = [], []
    for _ in range(k):
        m = jnp.max(vals, axis=0, keepdims=True)
        cand = jnp.where(vals == m, iota, width)
        idx = jnp.min(cand, axis=0, keepdims=True)
        out_v.append(m)
        out_i.append(idx)
        vals = jnp.where(iota == idx, -jnp.inf, vals)
    return out_v, out_i


def _l2n_lanes(v):
    n = jnp.sqrt(jnp.sum(v * v, axis=1, keepdims=True))
    return v / jnp.maximum(n, 1e-12)


def _l2n_subl(v):
    n = jnp.sqrt(jnp.sum(v * v, axis=0, keepdims=True))
    return v / jnp.maximum(n, 1e-12)


def _peer_body(x_ref, wqt_ref, bq_ref, g_ref, b_ref, sk0_ref, sk1_ref,
               ed_ref, eu_ref, o_ref):
    f32 = jnp.float32
    xb = x_ref[...]

    qT = jax.lax.dot_general(wqt_ref[...], xb, _NT,
                             preferred_element_type=f32) + bq_ref[...]

    sk0n = _l2n_lanes(sk0_ref[...])
    sk1n = _l2n_lanes(sk1_ref[...])

    dense_dT = jax.lax.dot_general(ed_ref[...], xb, _NT,
                                   preferred_element_type=f32)

    iota_e = jax.lax.broadcasted_iota(jnp.int32, (_EUSE, _TB), 0)
    iota8 = jax.lax.broadcasted_iota(jnp.int32, (_KP, _TB), 0)
    acc = jnp.zeros((_EUSE, _TB), f32)

    for h in range(_H):
        qhT = qT[h * _QD:(h + 1) * _QD, :]
        mu = jnp.mean(qhT, axis=0, keepdims=True)
        qc = qhT - mu
        var = jnp.mean(qc * qc, axis=0, keepdims=True)
        qn = qc * jax.lax.rsqrt(var + 1e-5) * g_ref[...] + b_ref[...]

        q0n = _l2n_subl(qn[:_SUBQ, :])
        q1n = _l2n_subl(qn[_SUBQ:, :])

        s0 = jax.lax.dot_general(sk0n, q0n, _NN,
                                 preferred_element_type=f32)
        s1 = jax.lax.dot_general(sk1n, q1n, _NN,
                                 preferred_element_type=f32)

        ts0, ti0 = _topk_rows(s0, _KP, _PK)
        ts1, ti1 = _topk_rows(s1, _KP, _PK)
        ts1c = jnp.concatenate(ts1, axis=0)
        ti0c = jnp.concatenate(ti0, axis=0)
        ti1c = jnp.concatenate(ti1, axis=0)

        comb = jnp.concatenate([ts0[a] + ts1c for a in range(_KP)],
                               axis=0)
        fs, fpos = _topk_rows(comb, _TOPK, _KP * _KP)

        m = fs[0]
        es = [jnp.exp(v - m) for v in fs]
        zden = es[0]
        for v in es[1:]:
            zden = zden + v
        inv_z = 1.0 / zden

        for k in range(_TOPK):
            pos = fpos[k]
            a = jax.lax.shift_right_logical(pos, 3)
            bsub = pos - (a << 3)
            i0 = jnp.sum(jnp.where(iota8 == a, ti0c, 0), axis=0,
                         keepdims=True)
            i1 = jnp.sum(jnp.where(iota8 == bsub, ti1c, 0), axis=0,
                         keepdims=True)
            eidx = i0 + i1
            gk = es[k] * inv_z

            onehot = eidx == iota_e
            hk = jnp.sum(jnp.where(onehot, dense_dT, 0.0), axis=0,
                         keepdims=True)
            hk = 0.5 * hk * (1.0 + jax.lax.erf(hk * 0.7071067811865476))
            hk = hk * gk
            acc = acc + jnp.where(onehot, hk, 0.0)

    o_ref[...] = jax.lax.dot_general(acc, eu_ref[...], (((0,), (0,)), ((), ())),
                                     preferred_element_type=f32)


@jax.jit
def kernel(hidden_states, Wq, bq, ln_gamma, ln_beta, sub_keys_0, sub_keys_1,
           expert_down, expert_up):
    x = hidden_states.reshape(_S, _D)
    out = pl.pallas_call(
        _peer_body,
        grid=(_S // _TB,),
        in_specs=[
            pl.BlockSpec((_TB, _D), lambda i: (i, 0)),
            pl.BlockSpec((_H * _QD, _D), lambda i: (0, 0)),
            pl.BlockSpec((_H * _QD, 1), lambda i: (0, 0)),
            pl.BlockSpec((_QD, 1), lambda i: (0, 0)),
            pl.BlockSpec((_QD, 1), lambda i: (0, 0)),
            pl.BlockSpec((_PK, _SUBQ), lambda i: (0, 0)),
            pl.BlockSpec((_PK, _SUBQ), lambda i: (0, 0)),
            pl.BlockSpec((_EUSE, _D), lambda i: (0, 0)),
            pl.BlockSpec((_EUSE, _D), lambda i: (0, 0)),
        ],
        out_specs=pl.BlockSpec((_TB, _D), lambda i: (i, 0)),
        out_shape=jax.ShapeDtypeStruct((_S, _D), jnp.float32),
    )(x, Wq.T, bq.reshape(_H * _QD, 1), ln_gamma.reshape(_QD, 1),
      ln_beta.reshape(_QD, 1), sub_keys_0, sub_keys_1, expert_down, expert_up)
    return out.reshape(_B, _S, _D)


# speedup vs baseline: 30.7454x; 1.0187x over previous
import jax
import jax.numpy as jnp
from jax.experimental import pallas as pl

_B, _S, _D = 1, 2048, 768
_H, _QD = 8, 256
_PK = 128
_SUBQ = _QD // 2
_TOPK = 8
_KP = 8
_EUSE = 256

_TB = 512

_NN = (((1,), (0,)), ((), ()))
_NT = (((1,), (1,)), ((), ()))


def _topk_rows(s, k, width):
    iota = jax.lax.broadcasted_iota(jnp.int32, s.shape, 0)
    vals = s
    out_v, out_i = [], []
    for _ in range(k):
        m = jnp.max(vals, axis=0, keepdims=True)
        cand = jnp.where(vals == m, iota, width)
        idx = jnp.min(cand, axis=0, keepdims=True)
        out_v.append(m)
        out_i.append(idx)
        vals = jnp.where(iota == idx, -jnp.inf, vals)
    return out_v, out_i


def _l2n_lanes(v):
    n = jnp.sqrt(jnp.sum(v * v, axis=1, keepdims=True))
    return v / jnp.maximum(n, 1e-12)


def _l2n_subl(v):
    n = jnp.sqrt(jnp.sum(v * v, axis=0, keepdims=True))
    return v / jnp.maximum(n, 1e-12)


def _peer_body(x_ref, wqt_ref, bq_ref, g_ref, b_ref, sk0_ref, sk1_ref,
               ed_ref, eu_ref, o_ref):
    f32 = jnp.float32
    xb = x_ref[...]

    qT = jax.lax.dot_general(wqt_ref[...], xb, _NT,
                             preferred_element_type=f32) + bq_ref[...]

    sk0n = _l2n_lanes(sk0_ref[...])
    sk1n = _l2n_lanes(sk1_ref[...])

    dense_dT = jax.lax.dot_general(ed_ref[...], xb, _NT,
                                   preferred_element_type=f32)

    iota_e = jax.lax.broadcasted_iota(jnp.int32, (_EUSE, _TB), 0)
    iota8 = jax.lax.broadcasted_iota(jnp.int32, (_KP, _TB), 0)
    acc = jnp.zeros((_EUSE, _TB), f32)

    for h in range(_H):
        qhT = qT[h * _QD:(h + 1) * _QD, :]
        mu = jnp.mean(qhT, axis=0, keepdims=True)
        qc = qhT - mu
        var = jnp.mean(qc * qc, axis=0, keepdims=True)
        qn = qc * jax.lax.rsqrt(var + 1e-5) * g_ref[...] + b_ref[...]

        q0n = _l2n_subl(qn[:_SUBQ, :])
        q1n = _l2n_subl(qn[_SUBQ:, :])

        s0 = jax.lax.dot_general(sk0n, q0n, _NN,
                                 preferred_element_type=f32)
        s1 = jax.lax.dot_general(sk1n, q1n, _NN,
                                 preferred_element_type=f32)

        ts0, ti0 = _topk_rows(s0, _KP, _PK)
        ts1, ti1 = _topk_rows(s1, _KP, _PK)
        ts1c = jnp.concatenate(ts1, axis=0)
        ti0c = jnp.concatenate(ti0, axis=0)
        ti1c = jnp.concatenate(ti1, axis=0)

        comb = jnp.concatenate([ts0[a] + ts1c for a in range(_KP)],
                               axis=0)
        fs, fpos = _topk_rows(comb, _TOPK, _KP * _KP)

        m = fs[0]
        es = [jnp.exp(v - m) for v in fs]
        zden = es[0]
        for v in es[1:]:
            zden = zden + v
        inv_z = 1.0 / zden

        for k in range(_TOPK):
            pos = fpos[k]
            a = jax.lax.shift_right_logical(pos, 3)
            bsub = pos - (a << 3)
            i0 = jnp.sum(jnp.where(iota8 == a, ti0c, 0), axis=0,
                         keepdims=True)
            i1 = jnp.sum(jnp.where(iota8 == bsub, ti1c, 0), axis=0,
                         keepdims=True)
            eidx = i0 + i1
            gk = es[k] * inv_z

            onehot = eidx == iota_e
            hk = jnp.sum(jnp.where(onehot, dense_dT, 0.0), axis=0,
                         keepdims=True)
            hk = 0.5 * hk * (1.0 + jax.lax.erf(hk * 0.7071067811865476))
            hk = hk * gk
            acc = acc + jnp.where(onehot, hk, 0.0)

    o_ref[...] = jax.lax.dot_general(acc, eu_ref[...], (((0,), (0,)), ((), ())),
                                     preferred_element_type=f32)


@jax.jit
def kernel(hidden_states, Wq, bq, ln_gamma, ln_beta, sub_keys_0, sub_keys_1,
           expert_down, expert_up):
    x = hidden_states.reshape(_S, _D)
    out = pl.pallas_call(
        _peer_body,
        grid=(_S // _TB,),
        in_specs=[
            pl.BlockSpec((_TB, _D), lambda i: (i, 0)),
            pl.BlockSpec((_H * _QD, _D), lambda i: (0, 0)),
            pl.BlockSpec((_H * _QD, 1), lambda i: (0, 0)),
            pl.BlockSpec((_QD, 1), lambda i: (0, 0)),
            pl.BlockSpec((_QD, 1), lambda i: (0, 0)),
            pl.BlockSpec((_PK, _SUBQ), lambda i: (0, 0)),
            pl.BlockSpec((_PK, _SUBQ), lambda i: (0, 0)),
            pl.BlockSpec((_EUSE, _D), lambda i: (0, 0)),
            pl.BlockSpec((_EUSE, _D), lambda i: (0, 0)),
        ],
        out_specs=pl.BlockSpec((_TB, _D), lambda i: (i, 0)),
        out_shape=jax.ShapeDtypeStruct((_S, _D), jnp.float32),
    )(x, Wq.T, bq.reshape(_H * _QD, 1), ln_gamma.reshape(_QD, 1),
      ln_beta.reshape(_QD, 1), sub_keys_0, sub_keys_1, expert_down, expert_up)
    return out.reshape(_B, _S, _D)
